# baseline (speedup 1.0000x reference)
import functools

import jax
import jax.numpy as jnp
from jax import lax
from jax.experimental import pallas as pl
from jax.experimental.pallas import tpu as pltpu
from jax.experimental.pallas import tpu_sc as plsc

_V = 1000000
_D = 64
_B = 16384
_NEG = 20
_SLOTS = _NEG + 2
_ROWS = _B * _SLOTS
_NC = 2
_NS = 16
_NW = _NC * _NS
_RPW = _ROWS // _NW
_CH = 128
_NCH = _RPW // _CH

_GRP = 8192
_HG = _GRP // 2
_PBK = -(-_V // _GRP)
_TROWS = _PBK * _HG

_BS = 512


def _repack(ut):
    def body(x1_ref, x2_ref, o_ref):
        o_ref[:, :_D] = jnp.transpose(
            x1_ref[...], (1, 0)).astype(jnp.bfloat16)
        o_ref[:, _D:] = jnp.transpose(
            x2_ref[...], (1, 0)).astype(jnp.bfloat16)

    return pl.pallas_call(
        body,
        grid=(_PBK,),
        in_specs=[
            pl.BlockSpec(
                (_D, _HG),
                lambda i: (0, jnp.minimum(2 * i, (_V - 1) // _HG))),
            pl.BlockSpec(
                (_D, _HG),
                lambda i: (0, jnp.minimum(2 * i + 1, (_V - 1) // _HG))),
        ],
        out_specs=pl.BlockSpec((_HG, 2 * _D), lambda i: (i, 0)),
        out_shape=jax.ShapeDtypeStruct((_TROWS, 2 * _D), jnp.bfloat16),
    )(ut, ut)


def _gather_rows(idx2d, table2):
    mesh = plsc.VectorSubcoreMesh(core_axis_name="c", subcore_axis_name="s")

    @functools.partial(
        pl.kernel,
        mesh=mesh,
        out_type=jax.ShapeDtypeStruct((_ROWS, 2 * _D), jnp.bfloat16),
        scratch_types=[
            pltpu.VMEM((_NCH, _CH), jnp.int32),
            pltpu.VMEM((_CH, 2 * _D), jnp.bfloat16),
            pltpu.SemaphoreType.DMA,
        ],
        compiler_params=pltpu.CompilerParams(use_tc_tiling_on_sc=False),
    )
    def k(idx_hbm, table_hbm, out_hbm, idx_v, rows_v, sem):
        wid = lax.axis_index("s") * _NC + lax.axis_index("c")
        pltpu.sync_copy(idx_hbm.at[pl.ds(wid * _NCH, _NCH)], idx_v)
        row0 = wid * _RPW

        def body(j, carry):
            pltpu.async_copy(table_hbm.at[idx_v.at[j]], rows_v, sem).wait()
            pltpu.sync_copy(rows_v, out_hbm.at[pl.ds(row0 + j * _CH, _CH)])
            return carry

        lax.fori_loop(0, _NCH, body, 0)

    return k(idx2d, table2)


def _score_body(neg_ref, u_ref, v_ref, pn_ref, pu_ref, pv_ref, d_ref, o_ref):
    i = pl.program_id(0)
    u128 = u_ref[0].astype(jnp.float32)
    puf = pu_ref[0].astype(jnp.float32)
    ue = u128[:, :_D] * (1.0 - puf) + u128[:, _D:] * puf
    ud = jnp.dot(ue, d_ref[...], preferred_element_type=jnp.float32)
    v128 = v_ref[0].astype(jnp.float32)
    pvf = pv_ref[0, :, 0].astype(jnp.float32)
    posE = jnp.sum(v128[:, :_D] * ud, axis=1)
    posO = jnp.sum(v128[:, _D:] * ud, axis=1)
    pos = posE * (1.0 - pvf) + posO * pvf
    nb = neg_ref[...].astype(jnp.float32)
    pnf = pn_ref[:, :, 0].astype(jnp.float32)
    negE = jnp.sum(nb[:, :, :_D] * ud[None, :, :], axis=2)
    negO = jnp.sum(nb[:, :, _D:] * ud[None, :, :], axis=2)
    neg = negE * (1.0 - pnf) + negO * pnf
    t = (jnp.sum(jax.nn.softplus(-jnp.clip(pos, -10.0, 10.0)))
         + jnp.sum(jax.nn.softplus(jnp.clip(neg, -10.0, 10.0)))) * (1.0 / _B)
    t2 = t[None, None]

    @pl.when(i == 0)
    def _():
        o_ref[...] = t2

    @pl.when(i > 0)
    def _():
        o_ref[...] += t2


def _score(g3, par3, diag):
    nblk = _B // _BS
    out = pl.pallas_call(
        _score_body,
        grid=(nblk,),
        in_specs=[
            pl.BlockSpec((_NEG, _BS, 2 * _D), lambda i: (0, i, 0)),
            pl.BlockSpec((1, _BS, 2 * _D), lambda i: (_NEG, i, 0)),
            pl.BlockSpec((1, _BS, 2 * _D), lambda i: (_NEG + 1, i, 0)),
            pl.BlockSpec((_NEG, _BS, 1), lambda i: (0, i, 0)),
            pl.BlockSpec((1, _BS, 1), lambda i: (_NEG, i, 0)),
            pl.BlockSpec((1, _BS, 1), lambda i: (_NEG + 1, i, 0)),
            pl.BlockSpec((_D, _D), lambda i: (0, 0)),
        ],
        out_specs=pl.BlockSpec((1, 1), lambda i: (0, 0)),
        out_shape=jax.ShapeDtypeStruct((1, 1), jnp.float32),
        compiler_params=pltpu.CompilerParams(
            vmem_limit_bytes=100 * 1024 * 1024),
    )(g3, g3, g3, par3, par3, par3, diag)
    return out[0, 0]


def kernel(pos_u, pos_v, neg_v, diag, u_weight):
    table2 = _repack(u_weight.T)
    idx = jnp.concatenate(
        [neg_v.T.reshape(-1), pos_u, pos_v]).astype(jnp.int32)
    par3 = ((idx >> 12) & 1).reshape(_SLOTS, _B, 1)
    rows = (((idx >> 13) << 12) | (idx & 4095)).reshape(_NW * _NCH, _CH)
    g = _gather_rows(rows, table2)
    g3 = g.reshape(_SLOTS, _B, 2 * _D)
    return _score(g3, par3, diag)


# speedup vs baseline: 1.8664x; 1.8664x over previous
import functools

import jax
import jax.numpy as jnp
from jax import lax
from jax.experimental import pallas as pl
from jax.experimental.pallas import tpu as pltpu
from jax.experimental.pallas import tpu_sc as plsc

_V = 1000000
_D = 64
_B = 16384
_NEG = 20
_SLOTS = _NEG + 2
_ROWS = _B * _SLOTS
_NC = 2
_NS = 16
_NW = _NC * _NS
_RPW = _ROWS // _NW
_CH = 128
_NCH = _RPW // _CH

_GRP = 16384
_HG = _GRP // 2
_PBK = -(-_V // _GRP)
_TROWS = _PBK * _HG

_BS = 512


def _repack(ut):
    def body(x1_ref, x2_ref, o_ref):
        o_ref[:, :_D] = jnp.transpose(x1_ref[...], (1, 0))
        o_ref[:, _D:] = jnp.transpose(x2_ref[...], (1, 0))

    return pl.pallas_call(
        body,
        grid=(_PBK,),
        in_specs=[
            pl.BlockSpec(
                (_D, _HG),
                lambda i: (0, jnp.minimum(2 * i, (_V - 1) // _HG))),
            pl.BlockSpec(
                (_D, _HG),
                lambda i: (0, jnp.minimum(2 * i + 1, (_V - 1) // _HG))),
        ],
        out_specs=pl.BlockSpec((_HG, 2 * _D), lambda i: (i, 0)),
        out_shape=jax.ShapeDtypeStruct((_TROWS, 2 * _D), jnp.float32),
    )(ut, ut)


def _gather_rows(idx2d, table2):
    mesh = plsc.VectorSubcoreMesh(core_axis_name="c", subcore_axis_name="s")

    @functools.partial(
        pl.kernel,
        mesh=mesh,
        out_type=jax.ShapeDtypeStruct((_ROWS, 2 * _D), jnp.float32),
        scratch_types=[
            pltpu.VMEM((_NCH, _CH), jnp.int32),
            pltpu.VMEM((_CH, 2 * _D), jnp.float32),
            pltpu.SemaphoreType.DMA,
        ],
        compiler_params=pltpu.CompilerParams(use_tc_tiling_on_sc=False),
    )
    def k(idx_hbm, table_hbm, out_hbm, idx_v, rows_v, sem):
        wid = lax.axis_index("s") * _NC + lax.axis_index("c")
        pltpu.sync_copy(idx_hbm.at[pl.ds(wid * _NCH, _NCH)], idx_v)
        row0 = wid * _RPW

        def body(j, carry):
            pltpu.async_copy(table_hbm.at[idx_v.at[j]], rows_v, sem).wait()
            pltpu.sync_copy(rows_v, out_hbm.at[pl.ds(row0 + j * _CH, _CH)])
            return carry

        lax.fori_loop(0, _NCH, body, 0)

    return k(idx2d, table2)


def _score_body(neg_ref, u_ref, v_ref, pn_ref, pu_ref, pv_ref, d_ref, o_ref):
    i = pl.program_id(0)
    lane = lax.broadcasted_iota(jnp.int32, (2 * _D, 2), 0)
    w2 = jnp.where((lane < _D) == (lax.broadcasted_iota(
        jnp.int32, (2 * _D, 2), 1) == 0), 1.0, 0.0)
    u128 = u_ref[0].astype(jnp.float32)
    puf = pu_ref[0].astype(jnp.float32)
    ue = u128[:, :_D] * (1.0 - puf) + u128[:, _D:] * puf
    ud = jnp.dot(ue, d_ref[...], preferred_element_type=jnp.float32)
    udud = jnp.concatenate([ud, ud], axis=1)
    v128 = v_ref[0].astype(jnp.float32)
    pvf = pv_ref[0, :, 0].astype(jnp.float32)
    sv = jnp.dot(v128 * udud, w2, preferred_element_type=jnp.float32)
    pos = sv[:, 0] + pvf * (sv[:, 1] - sv[:, 0])
    nb = neg_ref[...].astype(jnp.float32)
    pnf = pn_ref[:, :, 0].astype(jnp.float32).reshape(_NEG * _BS)
    q = (nb * udud[None, :, :]).reshape(_NEG * _BS, 2 * _D)
    sn = jnp.dot(q, w2, preferred_element_type=jnp.float32)
    neg = sn[:, 0] + pnf * (sn[:, 1] - sn[:, 0])
    t = (jnp.sum(jax.nn.softplus(-jnp.clip(pos, -10.0, 10.0)))
         + jnp.sum(jax.nn.softplus(jnp.clip(neg, -10.0, 10.0)))) * (1.0 / _B)
    t2 = t[None, None]

    @pl.when(i == 0)
    def _():
        o_ref[...] = t2

    @pl.when(i > 0)
    def _():
        o_ref[...] += t2


def _score(g3, par3, diag):
    nblk = _B // _BS
    out = pl.pallas_call(
        _score_body,
        grid=(nblk,),
        in_specs=[
            pl.BlockSpec((_NEG, _BS, 2 * _D), lambda i: (0, i, 0)),
            pl.BlockSpec((1, _BS, 2 * _D), lambda i: (_NEG, i, 0)),
            pl.BlockSpec((1, _BS, 2 * _D), lambda i: (_NEG + 1, i, 0)),
            pl.BlockSpec((_NEG, _BS, 1), lambda i: (0, i, 0)),
            pl.BlockSpec((1, _BS, 1), lambda i: (_NEG, i, 0)),
            pl.BlockSpec((1, _BS, 1), lambda i: (_NEG + 1, i, 0)),
            pl.BlockSpec((_D, _D), lambda i: (0, 0)),
        ],
        out_specs=pl.BlockSpec((1, 1), lambda i: (0, 0)),
        out_shape=jax.ShapeDtypeStruct((1, 1), jnp.float32),
        compiler_params=pltpu.CompilerParams(
            vmem_limit_bytes=100 * 1024 * 1024),
    )(g3, g3, g3, par3, par3, par3, diag)
    return out[0, 0]


def kernel(pos_u, pos_v, neg_v, diag, u_weight):
    table2 = _repack(u_weight.T)
    idx = jnp.concatenate(
        [neg_v.T.reshape(-1), pos_u, pos_v]).astype(jnp.int32)
    par3 = ((idx // _HG) & 1).reshape(_SLOTS, _B, 1)
    rows = ((idx // _GRP) * _HG + (idx & (_HG - 1))).reshape(
        _NW * _NCH, _CH)
    g = _gather_rows(rows, table2)
    g3 = g.reshape(_SLOTS, _B, 2 * _D)
    return _score(g3, par3, diag)


# speedup vs baseline: 3.2240x; 1.7274x over previous
import functools

import jax
import jax.numpy as jnp
from jax import lax
from jax.experimental import pallas as pl
from jax.experimental.pallas import tpu as pltpu
from jax.experimental.pallas import tpu_sc as plsc

_V = 1000000
_D = 64
_B = 16384
_NEG = 20
_SLOTS = _NEG + 2
_ROWS = _B * _SLOTS
_NC = 2
_NS = 16
_NW = _NC * _NS
_RPW = _ROWS // _NW
_CH = 128
_NCH = _RPW // _CH

_GRP = 16384
_HG = _GRP // 2
_PBK = -(-_V // _GRP)
_TROWS = _PBK * _HG

_HB = 512


def _repack(ut):
    def body(x1_ref, x2_ref, o_ref):
        o_ref[:, :_D] = jnp.transpose(x1_ref[...], (1, 0))
        o_ref[:, _D:] = jnp.transpose(x2_ref[...], (1, 0))

    return pl.pallas_call(
        body,
        grid=(_PBK,),
        in_specs=[
            pl.BlockSpec(
                (_D, _HG),
                lambda i: (0, jnp.minimum(2 * i, (_V - 1) // _HG))),
            pl.BlockSpec(
                (_D, _HG),
                lambda i: (0, jnp.minimum(2 * i + 1, (_V - 1) // _HG))),
        ],
        out_specs=pl.BlockSpec((_HG, 2 * _D), lambda i: (i, 0)),
        out_shape=jax.ShapeDtypeStruct((_TROWS, 2 * _D), jnp.float32),
    )(ut, ut)


def _gather_rows(idx2d, table2):
    mesh = plsc.VectorSubcoreMesh(core_axis_name="c", subcore_axis_name="s")

    @functools.partial(
        pl.kernel,
        mesh=mesh,
        out_type=jax.ShapeDtypeStruct((_ROWS, _D), jnp.float32),
        scratch_types=[
            pltpu.VMEM((_NCH, _CH), jnp.int32),
            pltpu.VMEM((_CH, _D), jnp.float32),
            pltpu.SemaphoreType.DMA,
        ],
        compiler_params=pltpu.CompilerParams(use_tc_tiling_on_sc=False),
    )
    def k(idx_hbm, table_hbm, out_hbm, idx_v, rows_v, sem):
        wid = lax.axis_index("s") * _NC + lax.axis_index("c")
        pltpu.sync_copy(idx_hbm.at[pl.ds(wid * _NCH, _NCH)], idx_v)
        row0 = wid * _RPW

        def body(j, carry):
            pltpu.async_copy(table_hbm.at[idx_v.at[j]], rows_v, sem).wait()
            pltpu.sync_copy(rows_v, out_hbm.at[pl.ds(row0 + j * _CH, _CH)])
            return carry

        lax.fori_loop(0, _NCH, body, 0)

    return k(idx2d, table2)


def _score_body(neg_ref, u_ref, v_ref, d_ref, o_ref):
    i = pl.program_id(0)
    lane = lax.broadcasted_iota(jnp.int32, (2 * _D, 2), 0)
    w2 = jnp.where((lane < _D) == (lax.broadcasted_iota(
        jnp.int32, (2 * _D, 2), 1) == 0), 1.0, 0.0)
    d = d_ref[...]
    ub = u_ref[0]
    udE = jnp.dot(ub[:, :_D], d, preferred_element_type=jnp.float32)
    udO = jnp.dot(ub[:, _D:], d, preferred_element_type=jnp.float32)
    udW = jnp.concatenate([udE, udO], axis=1)
    vb = v_ref[0]
    sv = jnp.dot(vb * udW, w2, preferred_element_type=jnp.float32)
    nb = neg_ref[...]
    q = (nb * udW[None, :, :]).reshape(_NEG * _HB, 2 * _D)
    sn = jnp.dot(q, w2, preferred_element_type=jnp.float32)
    t = (jnp.sum(jax.nn.softplus(-jnp.clip(sv, -10.0, 10.0)))
         + jnp.sum(jax.nn.softplus(jnp.clip(sn, -10.0, 10.0)))) * (1.0 / _B)
    t2 = t[None, None]

    @pl.when(i == 0)
    def _():
        o_ref[...] = t2

    @pl.when(i > 0)
    def _():
        o_ref[...] += t2


def _score(g3, diag):
    nblk = _B // (2 * _HB)
    out = pl.pallas_call(
        _score_body,
        grid=(nblk,),
        in_specs=[
            pl.BlockSpec((_NEG, _HB, 2 * _D), lambda i: (0, i, 0)),
            pl.BlockSpec((1, _HB, 2 * _D), lambda i: (_NEG, i, 0)),
            pl.BlockSpec((1, _HB, 2 * _D), lambda i: (_NEG + 1, i, 0)),
            pl.BlockSpec((_D, _D), lambda i: (0, 0)),
        ],
        out_specs=pl.BlockSpec((1, 1), lambda i: (0, 0)),
        out_shape=jax.ShapeDtypeStruct((1, 1), jnp.float32),
        compiler_params=pltpu.CompilerParams(
            vmem_limit_bytes=100 * 1024 * 1024),
    )(g3, g3, g3, diag)
    return out[0, 0]


def kernel(pos_u, pos_v, neg_v, diag, u_weight):
    table2 = _repack(u_weight.T)
    tv = table2.reshape(2 * _TROWS, _D)
    idx = jnp.concatenate(
        [neg_v.T.reshape(-1), pos_u, pos_v]).astype(jnp.int32)
    rows = (((idx // _GRP) * _HG + (idx & (_HG - 1))) * 2
            + ((idx // _HG) & 1)).reshape(_NW * _NCH, _CH)
    g = _gather_rows(rows, tv)
    g3 = g.reshape(_SLOTS, _B // 2, 2 * _D)
    return _score(g3, diag)
